# baseline (speedup 1.0000x reference)
import functools

import jax
import jax.numpy as jnp
from jax import lax
from jax.experimental import pallas as pl
from jax.experimental.pallas import tpu as pltpu
from jax.experimental.pallas import tpu_sc as plsc


def _sc_gather(idxT, tables, nc, ns):
    F, B = idxT.shape
    _, _, D = tables.shape
    nw = nc * ns
    bpw = B // nw
    mesh = plsc.VectorSubcoreMesh(core_axis_name="c", subcore_axis_name="s")

    @functools.partial(
        pl.kernel,
        mesh=mesh,
        out_type=jax.ShapeDtypeStruct((F, B, D), jnp.float32),
        scratch_types=[
            pltpu.VMEM((bpw,), jnp.int32),
            pltpu.VMEM((bpw, D), jnp.float32),
            pltpu.SemaphoreType.DMA,
        ],
        compiler_params=pltpu.CompilerParams(use_tc_tiling_on_sc=False),
    )
    def gather_kernel(idx_hbm, tab_hbm, x_hbm, idx_v, rows_v, sem):
        wid = lax.axis_index("s") * nc + lax.axis_index("c")
        base = wid * bpw

        def body(i, carry):
            pltpu.sync_copy(idx_hbm.at[i, pl.ds(base, bpw)], idx_v)
            pltpu.async_copy(tab_hbm.at[i].at[idx_v], rows_v, sem).wait()
            pltpu.sync_copy(rows_v, x_hbm.at[i, pl.ds(base, bpw)])
            return carry

        lax.fori_loop(0, F, body, 0)

    return gather_kernel(idxT, tables)


def _tc_matmul(x, mask, weight, bb=2048):
    F, B, D = x.shape
    A = weight.shape[-1]

    def body(x_ref, m_ref, w_ref, o_ref):
        acc = jnp.zeros((bb, A), jnp.float32)
        for i in range(F):
            wm = w_ref[i] * m_ref[i][:, None]
            acc = acc + jnp.dot(x_ref[i], wm,
                                preferred_element_type=jnp.float32)
        o_ref[...] = acc

    return pl.pallas_call(
        body,
        grid=(B // bb,),
        in_specs=[
            pl.BlockSpec((F, bb, D), lambda n: (0, n, 0)),
            pl.BlockSpec((F, D), lambda n: (0, 0)),
            pl.BlockSpec((F, D, A), lambda n: (0, 0, 0)),
        ],
        out_specs=pl.BlockSpec((bb, A), lambda n: (n, 0)),
        out_shape=jax.ShapeDtypeStruct((B, A), jnp.float32),
    )(x, mask, weight)


def kernel(inputs, tables, mask, weight):
    idxT = jnp.asarray(inputs, jnp.int32).T
    info = plsc.get_sparse_core_info()
    x = _sc_gather(idxT, tables, info.num_cores, info.num_subcores)
    return _tc_matmul(x, mask, weight)


# speedup vs baseline: 1.1482x; 1.1482x over previous
import functools

import jax
import jax.numpy as jnp
from jax import lax
from jax.experimental import pallas as pl
from jax.experimental.pallas import tpu as pltpu
from jax.experimental.pallas import tpu_sc as plsc

_LANES = 16
_NCHUNK = 8


def _sc_gather(idx_flat, offs, tab_flat, F, D, nc, ns):
    BF = idx_flat.shape[0]
    nw = nc * ns
    epw = BF // nw
    R = epw // _NCHUNK
    nvec = epw // _LANES
    mesh = plsc.VectorSubcoreMesh(core_axis_name="c", subcore_axis_name="s")

    @functools.partial(
        pl.kernel,
        mesh=mesh,
        out_type=jax.ShapeDtypeStruct((BF, D), jnp.float32),
        scratch_types=[
            pltpu.VMEM((epw,), jnp.int32),
            pltpu.VMEM((epw,), jnp.int32),
            pltpu.VMEM((R, D), jnp.float32),
            pltpu.VMEM((R, D), jnp.float32),
            pltpu.SemaphoreType.DMA,
            pltpu.SemaphoreType.DMA,
            pltpu.SemaphoreType.DMA,
            pltpu.SemaphoreType.DMA,
        ],
        compiler_params=pltpu.CompilerParams(use_tc_tiling_on_sc=False),
    )
    def gather_kernel(idx_hbm, offs_hbm, tab_hbm, x_hbm,
                      idx_v, offs_v, rows0, rows1, g0, g1, o0, o1):
        wid = lax.axis_index("s") * nc + lax.axis_index("c")
        base = wid * epw
        pltpu.sync_copy(idx_hbm.at[pl.ds(base, epw)], idx_v)
        pltpu.sync_copy(offs_hbm, offs_v)

        def add_body(k, carry):
            o = k * _LANES
            idx_v[pl.ds(o, _LANES)] = (
                idx_v[pl.ds(o, _LANES)] + offs_v[pl.ds(o, _LANES)])
            return carry

        lax.fori_loop(0, nvec, add_body, 0)

        rows = (rows0, rows1)
        gsem = (g0, g1)
        osem = (o0, o1)
        out_copies = [None, None]
        for c in range(_NCHUNK):
            b = c & 1
            if out_copies[b] is not None:
                out_copies[b].wait()
            gcp = pltpu.async_copy(
                tab_hbm.at[idx_v.at[pl.ds(c * R, R)]], rows[b], gsem[b])
            gcp.wait()
            ocp = pltpu.async_copy(
                rows[b], x_hbm.at[pl.ds(base + c * R, R)], osem[b])
            out_copies[b] = ocp
        for ocp in out_copies:
            ocp.wait()

    return gather_kernel(idx_flat, offs, tab_flat)


def _tc_matmul(x2, m2, wr, bb=2048):
    B, K = x2.shape
    A = wr.shape[-1]

    def body(x_ref, m_ref, w_ref, o_ref):
        wm = w_ref[...] * m_ref[...]
        o_ref[...] = jnp.dot(x_ref[...], wm,
                             preferred_element_type=jnp.float32)

    return pl.pallas_call(
        body,
        grid=(B // bb,),
        in_specs=[
            pl.BlockSpec((bb, K), lambda n: (n, 0)),
            pl.BlockSpec((K, 1), lambda n: (0, 0)),
            pl.BlockSpec((K, A), lambda n: (0, 0)),
        ],
        out_specs=pl.BlockSpec((bb, A), lambda n: (n, 0)),
        out_shape=jax.ShapeDtypeStruct((B, A), jnp.float32),
    )(x2, m2, wr)


def kernel(inputs, tables, mask, weight):
    B, F = inputs.shape
    _, V, D = tables.shape
    A = weight.shape[-1]
    info = plsc.get_sparse_core_info()
    nw = info.num_cores * info.num_subcores

    idx_flat = jnp.asarray(inputs, jnp.int32).reshape(B * F)
    offs = jnp.tile(jnp.arange(F, dtype=jnp.int32) * V, (B * F // nw) // F)
    tab_flat = tables.reshape(F * V, D)

    x = _sc_gather(idx_flat, offs, tab_flat, F, D,
                   info.num_cores, info.num_subcores)
    x2 = x.reshape(B, F * D)
    return _tc_matmul(x2, mask.reshape(F * D, 1), weight.reshape(F * D, A))
